# baseline (device time: 33247 ns/iter reference)
import functools

import jax
import jax.numpy as jnp
from jax import lax
from jax.experimental import pallas as pl
from jax.experimental.pallas import tpu as pltpu

N_DEV = 8
CAP = 25
E_PER = 4
BLK = E_PER * CAP


def kernel(x, router_W, route_idx, expert_W):
    del router_W
    n_tok, d_in = x.shape
    _, _, d_out = expert_W.shape
    n_exp = N_DEV * E_PER
    f32 = jnp.float32
    bf16 = jnp.bfloat16

    def body(x_hbm, idx_ref, w_hbm, out_ref,
             x_vmem, w_vmem, comm_ref, x_sem, w_sems,
             send_sems, recv_sems):
        my = lax.axis_index("i")
        others = [lax.rem(my + k, N_DEV) for k in range(1, N_DEV)]

        x_copy = pltpu.make_async_copy(x_hbm, x_vmem, x_sem)
        x_copy.start()
        w_copies = []
        for j in range(E_PER):
            c = pltpu.make_async_copy(w_hbm.at[j], w_vmem.at[j], w_sems.at[j])
            c.start()
            w_copies.append(c)

        barrier = pltpu.get_barrier_semaphore()
        for t in others:
            pl.semaphore_signal(
                barrier, inc=1,
                device_id=(t,), device_id_type=pl.DeviceIdType.MESH,
            )
        pl.semaphore_wait(barrier, N_DEV - 1)

        cols_e = lax.broadcasted_iota(jnp.int32, (n_tok, n_exp), 1)
        onehot = (idx_ref[:, :] == cols_e).astype(f32)
        pos = onehot
        d = 1
        while d < n_tok:
            shifted = jnp.concatenate(
                [jnp.zeros((d, n_exp), f32), pos[: n_tok - d, :]], axis=0
            )
            pos = pos + shifted
            d *= 2
        slot = pos * onehot
        slot = (slot * (slot <= CAP).astype(f32)).astype(bf16)

        er = lax.broadcasted_iota(jnp.int32, (n_exp, BLK), 0)
        cr = lax.broadcasted_iota(jnp.int32, (n_exp, BLK), 1)
        sel = (er == my * E_PER + cr // CAP).astype(bf16)
        mp = jnp.dot(slot, sel, preferred_element_type=f32)
        kp = (lax.broadcasted_iota(jnp.int32, (n_tok, BLK), 1) % CAP + 1
              ).astype(f32)
        m = (mp == kp).astype(bf16)
        x_copy.wait()
        cx = lax.dot_general(
            m, x_vmem[:, :].astype(bf16), (((0,), (0,)), ((), ())),
            preferred_element_type=f32,
        )

        sends = []
        for j in range(E_PER):
            w_copies[j].wait()
            comm_ref[my, j] = jnp.dot(
                cx[j * CAP:(j + 1) * CAP, :], w_vmem[j],
                preferred_element_type=f32,
            ).astype(bf16)
            for t in others:
                rdma = pltpu.make_async_remote_copy(
                    src_ref=comm_ref.at[my, j],
                    dst_ref=comm_ref.at[my, j],
                    send_sem=send_sems.at[t, j],
                    recv_sem=recv_sems.at[my, j],
                    device_id=(t,),
                    device_id_type=pl.DeviceIdType.MESH,
                )
                rdma.start()
                sends.append(rdma)

        er8 = lax.broadcasted_iota(jnp.int32, (n_exp, N_DEV * BLK), 0)
        cr8 = lax.broadcasted_iota(jnp.int32, (n_exp, N_DEV * BLK), 1)
        emat = (er8 == cr8 // CAP).astype(bf16)
        aexp = jnp.dot(slot, emat, preferred_element_type=f32)
        kp8 = (lax.broadcasted_iota(jnp.int32, (n_tok, N_DEV * BLK), 1)
               % CAP + 1).astype(f32)
        g = (aexp == kp8).astype(bf16)

        for t in others:
            for j in range(E_PER):
                recv = pltpu.make_async_remote_copy(
                    src_ref=comm_ref.at[t, j],
                    dst_ref=comm_ref.at[t, j],
                    send_sem=send_sems.at[t, j],
                    recv_sem=recv_sems.at[t, j],
                    device_id=(t,),
                    device_id_type=pl.DeviceIdType.MESH,
                )
                recv.wait_recv()

        gather = jnp.concatenate(
            [comm_ref[s, j] for s in range(N_DEV) for j in range(E_PER)],
            axis=0,
        )
        out_ref[:, :] = jnp.dot(g, gather, preferred_element_type=f32)

        for rdma in sends:
            rdma.wait_send()

        @functools.partial(
            pl.run_scoped, exit_barrier=pltpu.SemaphoreType.REGULAR
        )
        def _(exit_barrier):
            for t in others:
                pl.semaphore_signal(
                    exit_barrier, inc=1,
                    device_id=(t,), device_id_type=pl.DeviceIdType.MESH,
                )
            pl.semaphore_wait(exit_barrier, N_DEV - 1)

    return pl.pallas_call(
        body,
        out_shape=jax.ShapeDtypeStruct((n_tok, d_out), f32),
        in_specs=[
            pl.BlockSpec(memory_space=pl.ANY),
            pl.BlockSpec(memory_space=pltpu.VMEM),
            pl.BlockSpec(memory_space=pl.ANY),
        ],
        out_specs=pl.BlockSpec(memory_space=pltpu.VMEM),
        scratch_shapes=[
            pltpu.VMEM((n_tok, d_in), f32),
            pltpu.VMEM((E_PER, d_in, d_out), f32),
            pltpu.VMEM((N_DEV, E_PER, CAP, d_out), bf16),
            pltpu.SemaphoreType.DMA,
            pltpu.SemaphoreType.DMA((E_PER,)),
            pltpu.SemaphoreType.DMA((N_DEV, E_PER)),
            pltpu.SemaphoreType.DMA((N_DEV, E_PER)),
        ],
        compiler_params=pltpu.CompilerParams(collective_id=0),
    )(x, route_idx, expert_W)
